# baseline (speedup 1.0000x reference)
import functools

import numpy as np
import jax
import jax.numpy as jnp
from jax import lax
from jax.experimental import pallas as pl
from jax.experimental.pallas import tpu as pltpu
from jax.experimental.pallas import tpu_sc as plsc

NC = 2
NS = 16
LANES = 16
BATCH = 128
NBATCH = 6
SB = BATCH * NBATCH
FH = 16
_TC_R = 2048


def _round_up(a, m):
    return -(-a // m) * m


def _chunk_of(total, cap, align=1):
    return max(c for c in range(1, cap + 1)
               if total % c == 0 and c % align == 0)


def _mesh():
    return plsc.VectorSubcoreMesh(
        core_axis_name="c", subcore_axis_name="s", num_cores=NC, num_subcores=NS
    )


_SC_PARAMS = pltpu.CompilerParams(use_tc_tiling_on_sc=False)


@functools.lru_cache(maxsize=None)
def _make_deg_kernel(n_sc, e_pad):
    d_r = n_sc // NS
    acc_rows = n_sc
    sb_per_tile = e_pad // (NC * NS * SB)
    zc = _chunk_of(d_r, SB, align=8)

    @functools.partial(
        pl.kernel,
        out_type=jax.ShapeDtypeStruct((NC * acc_rows,), jnp.float32),
        mesh=_mesh(),
        compiler_params=_SC_PARAMS,
        scratch_types=[
            pltpu.VMEM_SHARED((acc_rows,), jnp.float32),
            pltpu.VMEM((NBATCH, BATCH), jnp.int32),
            pltpu.VMEM((NBATCH, BATCH), jnp.int32),
            pltpu.VMEM((SB,), jnp.float32),
            pltpu.VMEM((d_r,), jnp.float32),
            pltpu.SemaphoreType.DMA,
            pltpu.SemaphoreType.DMA,
        ],
    )
    def deg_kernel(dst_hbm, out_hbm, acc_sh, dstA, dstB, ones_v, obuf_v,
                   sem_i, sem_s):
        cid = lax.axis_index("c")
        sid = lax.axis_index("s")

        zeros16 = jnp.zeros((LANES,), jnp.float32)
        ones16 = jnp.ones((LANES,), jnp.float32)

        def fill0(i, _):
            obuf_v[pl.ds(i * LANES, LANES)] = zeros16
            return 0

        lax.fori_loop(0, d_r // LANES, fill0, 0)

        def fill1(i, _):
            ones_v[pl.ds(i * LANES, LANES)] = ones16
            return 0

        lax.fori_loop(0, SB // LANES, fill1, 0)

        for k in range(d_r // zc):
            pltpu.sync_copy(
                obuf_v.at[pl.ds(0, zc)],
                acc_sh.at[pl.ds(sid * d_r + k * zc, zc)],
            )
        plsc.subcore_barrier()

        row_base = (cid * NS + sid) * (sb_per_tile * NBATCH)
        nsb = sb_per_tile

        def fire_scatters(dst_v):
            for j in range(NBATCH):
                pltpu.async_copy(
                    ones_v.at[pl.ds(j * BATCH, BATCH)],
                    acc_sh.at[dst_v.at[j]],
                    sem_s,
                    add=True,
                )

        def wait_scatters(dst_v):
            for j in range(NBATCH):
                pltpu.make_async_copy(
                    ones_v.at[pl.ds(j * BATCH, BATCH)],
                    acc_sh.at[dst_v.at[j]],
                    sem_s,
                ).wait()

        pltpu.sync_copy(dst_hbm.at[pl.ds(row_base, NBATCH)], dstA)

        def one_iter(g, cur, prev):
            @pl.when(g > 0)
            def _():
                pltpu.make_async_copy(
                    dst_hbm.at[pl.ds(row_base, NBATCH)], cur, sem_i
                ).wait()

            fire_scatters(cur)

            @pl.when(g > 0)
            def _():
                wait_scatters(prev)

            @pl.when(g + 1 < nsb)
            def _():
                pltpu.async_copy(
                    dst_hbm.at[pl.ds(row_base + (g + 1) * NBATCH, NBATCH)],
                    prev,
                    sem_i,
                )

        def body(g, _):
            @pl.when(g % 2 == 0)
            def _():
                one_iter(g, dstA, dstB)

            @pl.when(g % 2 == 1)
            def _():
                one_iter(g, dstB, dstA)

            return 0

        lax.fori_loop(0, nsb, body, 0)
        wait_scatters(dstA if (nsb - 1) % 2 == 0 else dstB)
        plsc.subcore_barrier()

        pltpu.sync_copy(acc_sh.at[pl.ds(sid * d_r, d_r)], obuf_v)
        pltpu.sync_copy(obuf_v, out_hbm.at[pl.ds(cid * acc_rows + sid * d_r, d_r)])

    return deg_kernel


@functools.lru_cache(maxsize=None)
def _make_edge_pass(n_sc, e_pad, k_int, off):
    d_r = n_sc // NS
    out_rows = n_sc // NS
    sb_per_tile = e_pad // (NS * SB)
    zc = _chunk_of(d_r, SB, align=8)
    oc = _chunk_of(out_rows, SB, align=8)

    @functools.partial(
        pl.kernel,
        out_type=(
            jax.ShapeDtypeStruct((n_sc, FH), jnp.float32),
            jax.ShapeDtypeStruct((n_sc, FH), jnp.float32),
        ),
        mesh=_mesh(),
        compiler_params=_SC_PARAMS,
        scratch_types=[
            pltpu.VMEM_SHARED((n_sc, FH), jnp.float32),
            pltpu.VMEM((NBATCH, BATCH), jnp.int32),
            pltpu.VMEM((NBATCH, BATCH), jnp.int32),
            pltpu.VMEM((NBATCH, BATCH), jnp.int32),
            pltpu.VMEM((NBATCH, BATCH), jnp.int32),
            pltpu.VMEM((NBATCH, BATCH), jnp.int32),
            pltpu.VMEM((NBATCH, BATCH), jnp.int32),
            pltpu.VMEM((SB, FH), jnp.float32),
            pltpu.VMEM((SB, FH), jnp.float32),
            pltpu.SemaphoreType.DMA,
            pltpu.SemaphoreType.DMA,
            pltpu.SemaphoreType.DMA,
            pltpu.SemaphoreType.DMA,
        ],
    )
    def edge_pass(
        table, src_hbm, dst_hbm, out_lo, out_hi,
        acc_sh, srcA, dstA, gidxA, srcB, dstB, gidxB, rowsA, rowsB,
        sem_i, sem_g, sem_s, sem_o,
    ):
        cid = lax.axis_index("c")
        sid = lax.axis_index("s")
        qoff = off + cid

        zeros16 = jnp.zeros((LANES,), jnp.float32)

        def fill0(i, _):
            rowsA[i, pl.ds(0, LANES)] = zeros16
            return 0

        lax.fori_loop(0, SB, fill0, 0)
        for k in range(d_r // zc):
            pltpu.sync_copy(
                rowsA.at[pl.ds(0, zc)],
                acc_sh.at[pl.ds(sid * d_r + k * zc, zc)],
            )
        plsc.subcore_barrier()

        row_base = sid * (sb_per_tile * NBATCH)
        nsb = sb_per_tile

        def compute_gidx(src_v, gidx_v):
            for j in range(NBATCH):
                for q in range(BATCH // LANES):
                    s16 = src_v[j, pl.ds(q * LANES, LANES)]
                    gidx_v[j, pl.ds(q * LANES, LANES)] = s16 * k_int + qoff

        def fire_gathers(gidx_v, rows_v):
            for j in range(NBATCH):
                pltpu.async_copy(
                    table.at[gidx_v.at[j]],
                    rows_v.at[pl.ds(j * BATCH, BATCH)],
                    sem_g,
                )

        def wait_gathers(gidx_v, rows_v):
            for j in range(NBATCH):
                pltpu.make_async_copy(
                    table.at[gidx_v.at[j]],
                    rows_v.at[pl.ds(j * BATCH, BATCH)],
                    sem_g,
                ).wait()

        def fire_scatters(dst_v, rows_v):
            for j in range(NBATCH):
                pltpu.async_copy(
                    rows_v.at[pl.ds(j * BATCH, BATCH)],
                    acc_sh.at[dst_v.at[j]],
                    sem_s,
                    add=True,
                )

        def wait_scatters(dst_v, rows_v):
            for j in range(NBATCH):
                pltpu.make_async_copy(
                    rows_v.at[pl.ds(j * BATCH, BATCH)],
                    acc_sh.at[dst_v.at[j]],
                    sem_s,
                ).wait()

        pltpu.sync_copy(src_hbm.at[pl.ds(row_base, NBATCH)], srcA)
        pltpu.sync_copy(dst_hbm.at[pl.ds(row_base, NBATCH)], dstA)
        compute_gidx(srcA, gidxA)

        def one_iter(g, cur_gidx, cur_src, cur_dst, cur_rows,
                     prv_gidx, prv_src, prv_dst, prv_rows):
            @pl.when(g > 0)
            def _():
                pltpu.make_async_copy(
                    src_hbm.at[pl.ds(row_base, NBATCH)], cur_src, sem_i
                ).wait()
                pltpu.make_async_copy(
                    dst_hbm.at[pl.ds(row_base, NBATCH)], cur_dst, sem_i
                ).wait()
                compute_gidx(cur_src, cur_gidx)

            fire_gathers(cur_gidx, cur_rows)

            @pl.when(g > 0)
            def _():
                wait_gathers(prv_gidx, prv_rows)
                fire_scatters(prv_dst, prv_rows)
                wait_scatters(prv_dst, prv_rows)

            @pl.when(g + 1 < nsb)
            def _():
                rb1 = row_base + (g + 1) * NBATCH
                pltpu.async_copy(src_hbm.at[pl.ds(rb1, NBATCH)], prv_src, sem_i)
                pltpu.async_copy(dst_hbm.at[pl.ds(rb1, NBATCH)], prv_dst, sem_i)

        def body(g, _):
            @pl.when(g % 2 == 0)
            def _():
                one_iter(g, gidxA, srcA, dstA, rowsA, gidxB, srcB, dstB, rowsB)

            @pl.when(g % 2 == 1)
            def _():
                one_iter(g, gidxB, srcB, dstB, rowsB, gidxA, srcA, dstA, rowsA)

            return 0

        lax.fori_loop(0, nsb, body, 0)
        if (nsb - 1) % 2 == 0:
            lgidx, ldst, lrows = gidxA, dstA, rowsA
        else:
            lgidx, ldst, lrows = gidxB, dstB, rowsB
        wait_gathers(lgidx, lrows)
        fire_scatters(ldst, lrows)
        wait_scatters(ldst, lrows)
        plsc.subcore_barrier()

        def copy_out(out_hbm):
            nchunks = out_rows // oc
            for k in range(nchunks):
                rbuf = rowsA if k % 2 == 0 else rowsB
                if k >= 2:
                    pltpu.make_async_copy(
                        rbuf.at[pl.ds(0, oc)],
                        out_hbm.at[pl.ds(sid * out_rows, oc)],
                        sem_o,
                    ).wait()
                pltpu.sync_copy(
                    acc_sh.at[pl.ds(sid * out_rows + k * oc, oc)],
                    rbuf.at[pl.ds(0, oc)],
                )
                pltpu.async_copy(
                    rbuf.at[pl.ds(0, oc)],
                    out_hbm.at[pl.ds(sid * out_rows + k * oc, oc)],
                    sem_o,
                )
            for k in range(min(2, nchunks)):
                rbuf = rowsA if (nchunks - 2 + k) % 2 == 0 else rowsB
                pltpu.make_async_copy(
                    rbuf.at[pl.ds(0, oc)],
                    out_hbm.at[pl.ds(sid * out_rows, oc)],
                    sem_o,
                ).wait()

        @pl.when(cid == 0)
        def _():
            copy_out(out_lo)

        @pl.when(cid == 1)
        def _():
            copy_out(out_hi)

    return edge_pass


def _full(rows, cols):
    return pl.BlockSpec((rows, cols), lambda i: (0, 0))


def _blk(rows, cols):
    return pl.BlockSpec((rows, cols), lambda i: (i, 0))


def _kdinv_body(d0_ref, d1_ref, b16_ref, r16_ref):
    dinv = lax.rsqrt(d0_ref[...] + d1_ref[...] + 1.0)
    r16_ref[...] = jnp.dot(dinv, b16_ref[...],
                           preferred_element_type=jnp.float32, precision=lax.Precision.HIGHEST)


def _widen(r16, rep):
    pieces = []
    for a in range(8):
        t = r16[:, 16 * a : 16 * (a + 1)]
        pieces.extend([t] * rep)
    return jnp.concatenate(pieces, axis=1)


def _ka_body(x_ref, w_ref, r16_ref, g0_ref):
    g0_ref[...] = _widen(r16_ref[...], 2) * jnp.dot(
        x_ref[...], w_ref[...], preferred_element_type=jnp.float32, precision=lax.Precision.HIGHEST)


def _kb_body(alo_ref, ahi_ref, r16_ref, g0_ref, b_ref, w_ref,
             blo_ref, bhi_ref, h1_ref, g1_ref):
    r16 = r16_ref[...]
    accw = (jnp.dot(r16 * alo_ref[...], blo_ref[...],
                    preferred_element_type=jnp.float32, precision=lax.Precision.HIGHEST)
            + jnp.dot(r16 * ahi_ref[...], bhi_ref[...],
                      preferred_element_type=jnp.float32, precision=lax.Precision.HIGHEST))
    r32 = _widen(r16, 2)
    h1 = jnp.maximum(accw + r32 * g0_ref[...] + b_ref[...], 0.0)
    h1_ref[...] = h1
    g1_ref[...] = jnp.dot(r32 * h1, w_ref[...],
                          preferred_element_type=jnp.float32, precision=lax.Precision.HIGHEST)


def _kc_body(alo_ref, ahi_ref, r16_ref, g1_ref, h1_ref, b_ref,
             w_ref, blo_ref, bhi_ref, g2_ref):
    r16 = r16_ref[...]
    accw = (jnp.dot(r16 * alo_ref[...], blo_ref[...],
                    preferred_element_type=jnp.float32, precision=lax.Precision.HIGHEST)
            + jnp.dot(r16 * ahi_ref[...], bhi_ref[...],
                      preferred_element_type=jnp.float32, precision=lax.Precision.HIGHEST))
    r32 = _widen(r16, 2)
    h2 = (jnp.maximum(accw + r32 * g1_ref[...] + b_ref[...], 0.0)
          + h1_ref[...])
    g2_ref[...] = jnp.dot(r32 * h2, w_ref[...],
                          preferred_element_type=jnp.float32, precision=lax.Precision.HIGHEST)


def _kd_body(a0_ref, a1_ref, a2_ref, a3_ref, r16_ref, g2_ref,
             b_ref, p0_ref, p1_ref, p2_ref, p3_ref, out_ref):
    r16 = r16_ref[...]
    acc = jnp.dot(r16 * a0_ref[...], p0_ref[...],
                  preferred_element_type=jnp.float32, precision=lax.Precision.HIGHEST)
    acc = acc + jnp.dot(r16 * a1_ref[...], p1_ref[...],
                        preferred_element_type=jnp.float32, precision=lax.Precision.HIGHEST)
    acc = acc + jnp.dot(r16 * a2_ref[...], p2_ref[...],
                        preferred_element_type=jnp.float32, precision=lax.Precision.HIGHEST)
    acc = acc + jnp.dot(r16 * a3_ref[...], p3_ref[...],
                        preferred_element_type=jnp.float32, precision=lax.Precision.HIGHEST)
    out_ref[...] = acc + _widen(r16, 4) * g2_ref[...] + b_ref[...]


def kernel(x, edge_index, W0, b0, W1, b1, W_out, b_out):
    n, dfeat = x.shape
    e = edge_index.shape[1]
    nh = W0.shape[1]
    nclass = W_out.shape[1]
    grid_n = -(-n // _TC_R)
    n_sc = grid_n * _TC_R
    grid = (grid_n,)

    src = edge_index[0]
    dst = edge_index[1]
    e_pad = _round_up(e, NC * NS * SB)
    pad = e_pad - e
    src_p = jnp.concatenate([src, jnp.zeros((pad,), jnp.int32)]).reshape(-1, BATCH)
    dst_p = jnp.concatenate([dst, jnp.full((pad,), jnp.int32(n))]).reshape(-1, BATCH)

    m = np.arange(128)
    B16 = (m[:, None] == (np.arange(16 * 128) // 16)[None, :]).astype(np.float32)
    Blo = ((32 * (m // 16) + m % 16)[:, None]
           == np.arange(256)[None, :]).astype(np.float32)
    Bhi = ((32 * (m // 16) + 16 + m % 16)[:, None]
           == np.arange(256)[None, :]).astype(np.float32)
    B64 = [((64 * (m // 16) + 16 * j + m % 16)[:, None]
            == np.arange(512)[None, :]).astype(np.float32) for j in range(4)]

    W0bd = jnp.kron(jnp.eye(8, dtype=jnp.float32), W0)
    W1bd = jnp.kron(jnp.eye(8, dtype=jnp.float32), W1)
    Wobd = jnp.kron(jnp.eye(8, dtype=jnp.float32), W_out)
    b0w = jnp.tile(b0, 8)[None, :]
    b1w = jnp.tile(b1, 8)[None, :]
    bow = jnp.tile(b_out, 8)[None, :]

    deg_pp = _make_deg_kernel(n_sc, e_pad)(dst_p)
    d0 = deg_pp[:n_sc].reshape(n_sc // 128, 128)
    d1 = deg_pp[n_sc:].reshape(n_sc // 128, 128)

    pk1 = n_sc // 128
    pkf = n_sc * FH // 128

    kdinv = pl.pallas_call(
        _kdinv_body,
        grid=grid,
        in_specs=[_blk(16, 128), _blk(16, 128), _full(128, 2048)],
        out_specs=_blk(16, 2048),
        out_shape=jax.ShapeDtypeStruct((pk1, 2048), jnp.float32),
    )
    r16w = kdinv(d0, d1, B16)
    rep16 = r16w.reshape(pkf, 128)

    ka = pl.pallas_call(
        _ka_body,
        grid=grid,
        in_specs=[_blk(256, 1024), _full(1024, 256), _blk(256, 128)],
        out_specs=_blk(256, 256),
        out_shape=jax.ShapeDtypeStruct((n_sc // 8, 256), jnp.float32),
    )
    g0w = ka(x.reshape(n // 8, 8 * dfeat), W0bd, rep16)

    ep2 = _make_edge_pass(n_sc, e_pad, 2, 0)
    a0lo, a0hi = ep2(g0w.reshape(2 * n_sc, FH), src_p, dst_p)

    kb = pl.pallas_call(
        _kb_body,
        grid=grid,
        in_specs=[_blk(256, 128), _blk(256, 128), _blk(256, 128),
                  _blk(256, 256), _full(1, 256),
                  _full(256, 256), _full(128, 256), _full(128, 256)],
        out_specs=[_blk(256, 256), _blk(256, 256)],
        out_shape=[
            jax.ShapeDtypeStruct((n_sc // 8, 256), jnp.float32),
            jax.ShapeDtypeStruct((n_sc // 8, 256), jnp.float32),
        ],
    )
    h1w, g1w = kb(a0lo.reshape(pkf, 128), a0hi.reshape(pkf, 128), rep16,
                  g0w, b0w, W1bd, Blo, Bhi)

    a1lo, a1hi = ep2(g1w.reshape(2 * n_sc, FH), src_p, dst_p)

    kc = pl.pallas_call(
        _kc_body,
        grid=grid,
        in_specs=[_blk(256, 128), _blk(256, 128), _blk(256, 128),
                  _blk(256, 256), _blk(256, 256),
                  _full(1, 256), _full(256, 512), _full(128, 256),
                  _full(128, 256)],
        out_specs=_blk(256, 512),
        out_shape=jax.ShapeDtypeStruct((n_sc // 8, 512), jnp.float32),
    )
    g2w = kc(a1lo.reshape(pkf, 128), a1hi.reshape(pkf, 128), rep16,
             g1w, h1w, b1w, Wobd, Blo, Bhi)

    g2_tbl = g2w.reshape(4 * n_sc, FH)
    ep4a = _make_edge_pass(n_sc, e_pad, 4, 0)
    ep4b = _make_edge_pass(n_sc, e_pad, 4, 2)
    a2q0, a2q1 = ep4a(g2_tbl, src_p, dst_p)
    a2q2, a2q3 = ep4b(g2_tbl, src_p, dst_p)

    kd = pl.pallas_call(
        _kd_body,
        grid=grid,
        in_specs=[_blk(256, 128)] * 4 + [_blk(256, 128), _blk(256, 512),
                  _full(1, 512)]
                 + [_full(128, 512)] * 4,
        out_specs=_blk(256, 512),
        out_shape=jax.ShapeDtypeStruct((n_sc // 8, 512), jnp.float32),
    )
    outw = kd(a2q0.reshape(pkf, 128), a2q1.reshape(pkf, 128),
              a2q2.reshape(pkf, 128), a2q3.reshape(pkf, 128),
              rep16, g2w, bow, B64[0], B64[1], B64[2], B64[3])
    return outw.reshape(n_sc, nclass)[:n]


# speedup vs baseline: 1.1847x; 1.1847x over previous
import functools

import numpy as np
import jax
import jax.numpy as jnp
from jax import lax
from jax.experimental import pallas as pl
from jax.experimental.pallas import tpu as pltpu
from jax.experimental.pallas import tpu_sc as plsc

NC = 2
NS = 16
LANES = 16
BATCH = 128
NBATCH = 4
SB = BATCH * NBATCH
FH = 16
_TC_R = 2048


def _round_up(a, m):
    return -(-a // m) * m


def _chunk_of(total, cap, align=1):
    return max(c for c in range(1, cap + 1)
               if total % c == 0 and c % align == 0)


def _mesh():
    return plsc.VectorSubcoreMesh(
        core_axis_name="c", subcore_axis_name="s", num_cores=NC, num_subcores=NS
    )


_SC_PARAMS = pltpu.CompilerParams(use_tc_tiling_on_sc=False)


@functools.lru_cache(maxsize=None)
def _make_deg_kernel(n_sc, e_pad):
    d_r = n_sc // NS
    acc_rows = n_sc
    sb_per_tile = e_pad // (NC * NS * SB)
    zc = _chunk_of(d_r, SB, align=8)

    @functools.partial(
        pl.kernel,
        out_type=jax.ShapeDtypeStruct((NC * acc_rows,), jnp.float32),
        mesh=_mesh(),
        compiler_params=_SC_PARAMS,
        scratch_types=[
            pltpu.VMEM_SHARED((acc_rows,), jnp.float32),
            pltpu.VMEM((NBATCH, BATCH), jnp.int32),
            pltpu.VMEM((NBATCH, BATCH), jnp.int32),
            pltpu.VMEM((SB,), jnp.float32),
            pltpu.VMEM((d_r,), jnp.float32),
            pltpu.SemaphoreType.DMA,
            pltpu.SemaphoreType.DMA,
        ],
    )
    def deg_kernel(dst_hbm, out_hbm, acc_sh, dstA, dstB, ones_v, obuf_v,
                   sem_i, sem_s):
        cid = lax.axis_index("c")
        sid = lax.axis_index("s")

        zeros16 = jnp.zeros((LANES,), jnp.float32)
        ones16 = jnp.ones((LANES,), jnp.float32)

        def fill0(i, _):
            obuf_v[pl.ds(i * LANES, LANES)] = zeros16
            return 0

        lax.fori_loop(0, d_r // LANES, fill0, 0)

        def fill1(i, _):
            ones_v[pl.ds(i * LANES, LANES)] = ones16
            return 0

        lax.fori_loop(0, SB // LANES, fill1, 0)

        for k in range(d_r // zc):
            pltpu.sync_copy(
                obuf_v.at[pl.ds(0, zc)],
                acc_sh.at[pl.ds(sid * d_r + k * zc, zc)],
            )
        plsc.subcore_barrier()

        row_base = (cid * NS + sid) * (sb_per_tile * NBATCH)
        nsb = sb_per_tile

        def fire_scatters(dst_v):
            for j in range(NBATCH):
                pltpu.async_copy(
                    ones_v.at[pl.ds(j * BATCH, BATCH)],
                    acc_sh.at[dst_v.at[j]],
                    sem_s,
                    add=True,
                )

        def wait_scatters(dst_v):
            for j in range(NBATCH):
                pltpu.make_async_copy(
                    ones_v.at[pl.ds(j * BATCH, BATCH)],
                    acc_sh.at[dst_v.at[j]],
                    sem_s,
                ).wait()

        pltpu.sync_copy(dst_hbm.at[pl.ds(row_base, NBATCH)], dstA)

        def one_iter(g, cur, prev):
            @pl.when(g > 0)
            def _():
                pltpu.make_async_copy(
                    dst_hbm.at[pl.ds(row_base, NBATCH)], cur, sem_i
                ).wait()

            fire_scatters(cur)

            @pl.when(g > 0)
            def _():
                wait_scatters(prev)

            @pl.when(g + 1 < nsb)
            def _():
                pltpu.async_copy(
                    dst_hbm.at[pl.ds(row_base + (g + 1) * NBATCH, NBATCH)],
                    prev,
                    sem_i,
                )

        def body(g, _):
            @pl.when(g % 2 == 0)
            def _():
                one_iter(g, dstA, dstB)

            @pl.when(g % 2 == 1)
            def _():
                one_iter(g, dstB, dstA)

            return 0

        lax.fori_loop(0, nsb, body, 0)
        wait_scatters(dstA if (nsb - 1) % 2 == 0 else dstB)
        plsc.subcore_barrier()

        pltpu.sync_copy(acc_sh.at[pl.ds(sid * d_r, d_r)], obuf_v)
        pltpu.sync_copy(obuf_v, out_hbm.at[pl.ds(cid * acc_rows + sid * d_r, d_r)])

    return deg_kernel


@functools.lru_cache(maxsize=None)
def _make_edge_pass(n_sc, e_pad, k_int, off):
    d_r = n_sc // NS
    out_rows = n_sc // NS
    sb_per_tile = e_pad // (NS * SB)
    zc = _chunk_of(d_r, SB, align=8)
    oc = _chunk_of(out_rows, SB, align=8)

    @functools.partial(
        pl.kernel,
        out_type=(
            jax.ShapeDtypeStruct((n_sc, FH), jnp.float32),
            jax.ShapeDtypeStruct((n_sc, FH), jnp.float32),
        ),
        mesh=_mesh(),
        compiler_params=_SC_PARAMS,
        scratch_types=[
            pltpu.VMEM_SHARED((n_sc, FH), jnp.float32),
            pltpu.VMEM((NBATCH, BATCH), jnp.int32),
            pltpu.VMEM((NBATCH, BATCH), jnp.int32),
            pltpu.VMEM((NBATCH, BATCH), jnp.int32),
            pltpu.VMEM((NBATCH, BATCH), jnp.int32),
            pltpu.VMEM((NBATCH, BATCH), jnp.int32),
            pltpu.VMEM((NBATCH, BATCH), jnp.int32),
            pltpu.VMEM((SB, FH), jnp.float32),
            pltpu.VMEM((SB, FH), jnp.float32),
            pltpu.SemaphoreType.DMA,
            pltpu.SemaphoreType.DMA,
            pltpu.SemaphoreType.DMA,
            pltpu.SemaphoreType.DMA,
        ],
    )
    def edge_pass(
        table, src_hbm, dst_hbm, out_lo, out_hi,
        acc_sh, srcA, dstA, gidxA, srcB, dstB, gidxB, rowsA, rowsB,
        sem_i, sem_g, sem_s, sem_o,
    ):
        cid = lax.axis_index("c")
        sid = lax.axis_index("s")
        qoff = off + cid

        zeros16 = jnp.zeros((LANES,), jnp.float32)

        def fill0(i, _):
            rowsA[i, pl.ds(0, LANES)] = zeros16
            return 0

        lax.fori_loop(0, SB, fill0, 0)
        for k in range(d_r // zc):
            pltpu.sync_copy(
                rowsA.at[pl.ds(0, zc)],
                acc_sh.at[pl.ds(sid * d_r + k * zc, zc)],
            )
        plsc.subcore_barrier()

        row_base = sid * (sb_per_tile * NBATCH)
        nsb = sb_per_tile

        def compute_gidx(src_v, gidx_v):
            for j in range(NBATCH):
                for q in range(BATCH // LANES):
                    s16 = src_v[j, pl.ds(q * LANES, LANES)]
                    gidx_v[j, pl.ds(q * LANES, LANES)] = s16 * k_int + qoff

        def fire_gathers(gidx_v, rows_v):
            for j in range(NBATCH):
                pltpu.async_copy(
                    table.at[gidx_v.at[j]],
                    rows_v.at[pl.ds(j * BATCH, BATCH)],
                    sem_g,
                )

        def wait_gathers(gidx_v, rows_v):
            for j in range(NBATCH):
                pltpu.make_async_copy(
                    table.at[gidx_v.at[j]],
                    rows_v.at[pl.ds(j * BATCH, BATCH)],
                    sem_g,
                ).wait()

        def fire_scatters(dst_v, rows_v):
            for j in range(NBATCH):
                pltpu.async_copy(
                    rows_v.at[pl.ds(j * BATCH, BATCH)],
                    acc_sh.at[dst_v.at[j]],
                    sem_s,
                    add=True,
                )

        def wait_scatters(dst_v, rows_v):
            for j in range(NBATCH):
                pltpu.make_async_copy(
                    rows_v.at[pl.ds(j * BATCH, BATCH)],
                    acc_sh.at[dst_v.at[j]],
                    sem_s,
                ).wait()

        pltpu.sync_copy(src_hbm.at[pl.ds(row_base, NBATCH)], srcA)
        pltpu.sync_copy(dst_hbm.at[pl.ds(row_base, NBATCH)], dstA)
        compute_gidx(srcA, gidxA)

        def one_iter(g, cur_gidx, cur_src, cur_dst, cur_rows,
                     prv_gidx, prv_src, prv_dst, prv_rows):
            @pl.when(g > 0)
            def _():
                pltpu.make_async_copy(
                    src_hbm.at[pl.ds(row_base, NBATCH)], cur_src, sem_i
                ).wait()
                pltpu.make_async_copy(
                    dst_hbm.at[pl.ds(row_base, NBATCH)], cur_dst, sem_i
                ).wait()
                compute_gidx(cur_src, cur_gidx)

            fire_gathers(cur_gidx, cur_rows)

            @pl.when(g > 0)
            def _():
                wait_gathers(prv_gidx, prv_rows)
                fire_scatters(prv_dst, prv_rows)
                wait_scatters(prv_dst, prv_rows)

            @pl.when(g + 1 < nsb)
            def _():
                rb1 = row_base + (g + 1) * NBATCH
                pltpu.async_copy(src_hbm.at[pl.ds(rb1, NBATCH)], prv_src, sem_i)
                pltpu.async_copy(dst_hbm.at[pl.ds(rb1, NBATCH)], prv_dst, sem_i)

        def body(g, _):
            @pl.when(g % 2 == 0)
            def _():
                one_iter(g, gidxA, srcA, dstA, rowsA, gidxB, srcB, dstB, rowsB)

            @pl.when(g % 2 == 1)
            def _():
                one_iter(g, gidxB, srcB, dstB, rowsB, gidxA, srcA, dstA, rowsA)

            return 0

        lax.fori_loop(0, nsb, body, 0)
        if (nsb - 1) % 2 == 0:
            lgidx, ldst, lrows = gidxA, dstA, rowsA
        else:
            lgidx, ldst, lrows = gidxB, dstB, rowsB
        wait_gathers(lgidx, lrows)
        fire_scatters(ldst, lrows)
        wait_scatters(ldst, lrows)
        plsc.subcore_barrier()

        def copy_out(out_hbm):
            nchunks = out_rows // oc
            for k in range(nchunks):
                rbuf = rowsA if k % 2 == 0 else rowsB
                if k >= 2:
                    pltpu.make_async_copy(
                        rbuf.at[pl.ds(0, oc)],
                        out_hbm.at[pl.ds(sid * out_rows, oc)],
                        sem_o,
                    ).wait()
                pltpu.sync_copy(
                    acc_sh.at[pl.ds(sid * out_rows + k * oc, oc)],
                    rbuf.at[pl.ds(0, oc)],
                )
                pltpu.async_copy(
                    rbuf.at[pl.ds(0, oc)],
                    out_hbm.at[pl.ds(sid * out_rows + k * oc, oc)],
                    sem_o,
                )
            for k in range(min(2, nchunks)):
                rbuf = rowsA if (nchunks - 2 + k) % 2 == 0 else rowsB
                pltpu.make_async_copy(
                    rbuf.at[pl.ds(0, oc)],
                    out_hbm.at[pl.ds(sid * out_rows, oc)],
                    sem_o,
                ).wait()

        @pl.when(cid == 0)
        def _():
            copy_out(out_lo)

        @pl.when(cid == 1)
        def _():
            copy_out(out_hi)

    return edge_pass


def _full(rows, cols):
    return pl.BlockSpec((rows, cols), lambda i: (0, 0))


def _blk(rows, cols):
    return pl.BlockSpec((rows, cols), lambda i: (i, 0))


def _kdinv_body(d0_ref, d1_ref, b16_ref, r16_ref):
    dinv = lax.rsqrt(d0_ref[...] + d1_ref[...] + 1.0)
    r16_ref[...] = jnp.dot(dinv, b16_ref[...],
                           preferred_element_type=jnp.float32, precision=lax.Precision.HIGHEST)


def _widen(r16, rep):
    pieces = []
    for a in range(8):
        t = r16[:, 16 * a : 16 * (a + 1)]
        pieces.extend([t] * rep)
    return jnp.concatenate(pieces, axis=1)


def _ka_body(x_ref, w_ref, r16_ref, g0_ref):
    g0_ref[...] = _widen(r16_ref[...], 2) * jnp.dot(
        x_ref[...], w_ref[...], preferred_element_type=jnp.float32, precision=lax.Precision.HIGHEST)


def _kb_body(alo_ref, ahi_ref, r16_ref, g0_ref, b_ref, w_ref,
             blo_ref, bhi_ref, h1_ref, g1_ref):
    r16 = r16_ref[...]
    accw = (jnp.dot(r16 * alo_ref[...], blo_ref[...],
                    preferred_element_type=jnp.float32, precision=lax.Precision.HIGHEST)
            + jnp.dot(r16 * ahi_ref[...], bhi_ref[...],
                      preferred_element_type=jnp.float32, precision=lax.Precision.HIGHEST))
    r32 = _widen(r16, 2)
    h1 = jnp.maximum(accw + r32 * g0_ref[...] + b_ref[...], 0.0)
    h1_ref[...] = h1
    g1_ref[...] = jnp.dot(r32 * h1, w_ref[...],
                          preferred_element_type=jnp.float32, precision=lax.Precision.HIGHEST)


def _kc_body(alo_ref, ahi_ref, r16_ref, g1_ref, h1_ref, b_ref,
             w_ref, blo_ref, bhi_ref, g2_ref):
    r16 = r16_ref[...]
    accw = (jnp.dot(r16 * alo_ref[...], blo_ref[...],
                    preferred_element_type=jnp.float32, precision=lax.Precision.HIGHEST)
            + jnp.dot(r16 * ahi_ref[...], bhi_ref[...],
                      preferred_element_type=jnp.float32, precision=lax.Precision.HIGHEST))
    r32 = _widen(r16, 2)
    h2 = (jnp.maximum(accw + r32 * g1_ref[...] + b_ref[...], 0.0)
          + h1_ref[...])
    g2_ref[...] = jnp.dot(r32 * h2, w_ref[...],
                          preferred_element_type=jnp.float32, precision=lax.Precision.HIGHEST)


def _kd_body(a0_ref, a1_ref, a2_ref, a3_ref, r16_ref, g2_ref,
             b_ref, p0_ref, p1_ref, p2_ref, p3_ref, out_ref):
    r16 = r16_ref[...]
    acc = jnp.dot(r16 * a0_ref[...], p0_ref[...],
                  preferred_element_type=jnp.float32, precision=lax.Precision.HIGHEST)
    acc = acc + jnp.dot(r16 * a1_ref[...], p1_ref[...],
                        preferred_element_type=jnp.float32, precision=lax.Precision.HIGHEST)
    acc = acc + jnp.dot(r16 * a2_ref[...], p2_ref[...],
                        preferred_element_type=jnp.float32, precision=lax.Precision.HIGHEST)
    acc = acc + jnp.dot(r16 * a3_ref[...], p3_ref[...],
                        preferred_element_type=jnp.float32, precision=lax.Precision.HIGHEST)
    out_ref[...] = acc + _widen(r16, 4) * g2_ref[...] + b_ref[...]


def kernel(x, edge_index, W0, b0, W1, b1, W_out, b_out):
    n, dfeat = x.shape
    e = edge_index.shape[1]
    nh = W0.shape[1]
    nclass = W_out.shape[1]
    grid_n = -(-n // _TC_R)
    n_sc = grid_n * _TC_R
    grid = (grid_n,)

    src = edge_index[0]
    dst = edge_index[1]
    e_pad = _round_up(e, NC * NS * SB)
    pad = e_pad - e
    src_p = jnp.concatenate([src, jnp.zeros((pad,), jnp.int32)]).reshape(-1, BATCH)
    dst_p = jnp.concatenate([dst, jnp.full((pad,), jnp.int32(n))]).reshape(-1, BATCH)

    m = np.arange(128)
    B16 = (m[:, None] == (np.arange(16 * 128) // 16)[None, :]).astype(np.float32)
    Blo = ((32 * (m // 16) + m % 16)[:, None]
           == np.arange(256)[None, :]).astype(np.float32)
    Bhi = ((32 * (m // 16) + 16 + m % 16)[:, None]
           == np.arange(256)[None, :]).astype(np.float32)
    B64 = [((64 * (m // 16) + 16 * j + m % 16)[:, None]
            == np.arange(512)[None, :]).astype(np.float32) for j in range(4)]

    W0bd = jnp.kron(jnp.eye(8, dtype=jnp.float32), W0)
    W1bd = jnp.kron(jnp.eye(8, dtype=jnp.float32), W1)
    Wobd = jnp.kron(jnp.eye(8, dtype=jnp.float32), W_out)
    b0w = jnp.tile(b0, 8)[None, :]
    b1w = jnp.tile(b1, 8)[None, :]
    bow = jnp.tile(b_out, 8)[None, :]

    deg_pp = _make_deg_kernel(n_sc, e_pad)(dst_p)
    d0 = deg_pp[:n_sc].reshape(n_sc // 128, 128)
    d1 = deg_pp[n_sc:].reshape(n_sc // 128, 128)

    pk1 = n_sc // 128
    pkf = n_sc * FH // 128

    kdinv = pl.pallas_call(
        _kdinv_body,
        grid=grid,
        in_specs=[_blk(16, 128), _blk(16, 128), _full(128, 2048)],
        out_specs=_blk(16, 2048),
        out_shape=jax.ShapeDtypeStruct((pk1, 2048), jnp.float32),
    )
    r16w = kdinv(d0, d1, B16)
    rep16 = r16w.reshape(pkf, 128)

    ka = pl.pallas_call(
        _ka_body,
        grid=grid,
        in_specs=[_blk(256, 1024), _full(1024, 256), _blk(256, 128)],
        out_specs=_blk(256, 256),
        out_shape=jax.ShapeDtypeStruct((n_sc // 8, 256), jnp.float32),
    )
    g0w = ka(x.reshape(n // 8, 8 * dfeat), W0bd, rep16)

    ep2 = _make_edge_pass(n_sc, e_pad, 2, 0)
    a0lo, a0hi = ep2(g0w.reshape(2 * n_sc, FH), src_p, dst_p)

    kb = pl.pallas_call(
        _kb_body,
        grid=grid,
        in_specs=[_blk(256, 128), _blk(256, 128), _blk(256, 128),
                  _blk(256, 256), _full(1, 256),
                  _full(256, 256), _full(128, 256), _full(128, 256)],
        out_specs=[_blk(256, 256), _blk(256, 256)],
        out_shape=[
            jax.ShapeDtypeStruct((n_sc // 8, 256), jnp.float32),
            jax.ShapeDtypeStruct((n_sc // 8, 256), jnp.float32),
        ],
    )
    h1w, g1w = kb(a0lo.reshape(pkf, 128), a0hi.reshape(pkf, 128), rep16,
                  g0w, b0w, W1bd, Blo, Bhi)

    a1lo, a1hi = ep2(g1w.reshape(2 * n_sc, FH), src_p, dst_p)

    kc = pl.pallas_call(
        _kc_body,
        grid=grid,
        in_specs=[_blk(256, 128), _blk(256, 128), _blk(256, 128),
                  _blk(256, 256), _blk(256, 256),
                  _full(1, 256), _full(256, 512), _full(128, 256),
                  _full(128, 256)],
        out_specs=_blk(256, 512),
        out_shape=jax.ShapeDtypeStruct((n_sc // 8, 512), jnp.float32),
    )
    g2w = kc(a1lo.reshape(pkf, 128), a1hi.reshape(pkf, 128), rep16,
             g1w, h1w, b1w, Wobd, Blo, Bhi)

    g2_tbl = g2w.reshape(4 * n_sc, FH)
    ep4a = _make_edge_pass(n_sc, e_pad, 4, 0)
    ep4b = _make_edge_pass(n_sc, e_pad, 4, 2)
    a2q0, a2q1 = ep4a(g2_tbl, src_p, dst_p)
    a2q2, a2q3 = ep4b(g2_tbl, src_p, dst_p)

    kd = pl.pallas_call(
        _kd_body,
        grid=grid,
        in_specs=[_blk(256, 128)] * 4 + [_blk(256, 128), _blk(256, 512),
                  _full(1, 512)]
                 + [_full(128, 512)] * 4,
        out_specs=_blk(256, 512),
        out_shape=jax.ShapeDtypeStruct((n_sc // 8, 512), jnp.float32),
    )
    outw = kd(a2q0.reshape(pkf, 128), a2q1.reshape(pkf, 128),
              a2q2.reshape(pkf, 128), a2q3.reshape(pkf, 128),
              rep16, g2w, bow, B64[0], B64[1], B64[2], B64[3])
    return outw.reshape(n_sc, nclass)[:n]


# speedup vs baseline: 1.3280x; 1.1209x over previous
import functools

import numpy as np
import jax
import jax.numpy as jnp
from jax import lax
from jax.experimental import pallas as pl
from jax.experimental.pallas import tpu as pltpu
from jax.experimental.pallas import tpu_sc as plsc

NC = 2
NS = 16
LANES = 16
BATCH = 128
NBATCH = 4
SB = BATCH * NBATCH
FH = 16
_TC_R = 2048


def _round_up(a, m):
    return -(-a // m) * m


def _chunk_of(total, cap, align=1):
    return max(c for c in range(1, cap + 1)
               if total % c == 0 and c % align == 0)


def _mesh():
    return plsc.VectorSubcoreMesh(
        core_axis_name="c", subcore_axis_name="s", num_cores=NC, num_subcores=NS
    )


_SC_PARAMS = pltpu.CompilerParams(use_tc_tiling_on_sc=False)


@functools.lru_cache(maxsize=None)
def _make_deg_kernel(n_sc, e_pad):
    d_r = n_sc // NS
    acc_rows = n_sc
    sb_per_tile = e_pad // (NC * NS * SB)
    zc = _chunk_of(d_r, SB, align=8)

    @functools.partial(
        pl.kernel,
        out_type=jax.ShapeDtypeStruct((NC * acc_rows,), jnp.float32),
        mesh=_mesh(),
        compiler_params=_SC_PARAMS,
        scratch_types=[
            pltpu.VMEM_SHARED((acc_rows,), jnp.float32),
            pltpu.VMEM((NBATCH, BATCH), jnp.int32),
            pltpu.VMEM((NBATCH, BATCH), jnp.int32),
            pltpu.VMEM((SB,), jnp.float32),
            pltpu.VMEM((d_r,), jnp.float32),
            pltpu.SemaphoreType.DMA,
            pltpu.SemaphoreType.DMA,
        ],
    )
    def deg_kernel(dst_hbm, out_hbm, acc_sh, dstA, dstB, ones_v, obuf_v,
                   sem_i, sem_s):
        cid = lax.axis_index("c")
        sid = lax.axis_index("s")

        zeros16 = jnp.zeros((LANES,), jnp.float32)
        ones16 = jnp.ones((LANES,), jnp.float32)

        def fill0(i, _):
            obuf_v[pl.ds(i * LANES, LANES)] = zeros16
            return 0

        lax.fori_loop(0, d_r // LANES, fill0, 0)

        def fill1(i, _):
            ones_v[pl.ds(i * LANES, LANES)] = ones16
            return 0

        lax.fori_loop(0, SB // LANES, fill1, 0)

        for k in range(d_r // zc):
            pltpu.sync_copy(
                obuf_v.at[pl.ds(0, zc)],
                acc_sh.at[pl.ds(sid * d_r + k * zc, zc)],
            )
        plsc.subcore_barrier()

        row_base = (cid * NS + sid) * (sb_per_tile * NBATCH)
        nsb = sb_per_tile

        def fire_scatters(dst_v):
            for j in range(NBATCH):
                pltpu.async_copy(
                    ones_v.at[pl.ds(j * BATCH, BATCH)],
                    acc_sh.at[dst_v.at[j]],
                    sem_s,
                    add=True,
                )

        def wait_scatters(dst_v):
            for j in range(NBATCH):
                pltpu.make_async_copy(
                    ones_v.at[pl.ds(j * BATCH, BATCH)],
                    acc_sh.at[dst_v.at[j]],
                    sem_s,
                ).wait()

        pltpu.sync_copy(dst_hbm.at[pl.ds(row_base, NBATCH)], dstA)

        def one_iter(g, cur, prev):
            @pl.when(g > 0)
            def _():
                pltpu.make_async_copy(
                    dst_hbm.at[pl.ds(row_base, NBATCH)], cur, sem_i
                ).wait()

            fire_scatters(cur)

            @pl.when(g > 0)
            def _():
                wait_scatters(prev)

            @pl.when(g + 1 < nsb)
            def _():
                pltpu.async_copy(
                    dst_hbm.at[pl.ds(row_base + (g + 1) * NBATCH, NBATCH)],
                    prev,
                    sem_i,
                )

        def body(g, _):
            @pl.when(g % 2 == 0)
            def _():
                one_iter(g, dstA, dstB)

            @pl.when(g % 2 == 1)
            def _():
                one_iter(g, dstB, dstA)

            return 0

        lax.fori_loop(0, nsb, body, 0)
        wait_scatters(dstA if (nsb - 1) % 2 == 0 else dstB)
        plsc.subcore_barrier()

        pltpu.sync_copy(acc_sh.at[pl.ds(sid * d_r, d_r)], obuf_v)
        pltpu.sync_copy(obuf_v, out_hbm.at[pl.ds(cid * acc_rows + sid * d_r, d_r)])

    return deg_kernel


@functools.lru_cache(maxsize=None)
def _make_edge_pass(n_sc, e_pad, k_int, off):
    d_r = n_sc // NS
    out_rows = n_sc // NS
    sb_per_tile = e_pad // (NS * SB)
    zc = _chunk_of(d_r, SB, align=8)
    oc = _chunk_of(out_rows, SB, align=8)

    @functools.partial(
        pl.kernel,
        out_type=(
            jax.ShapeDtypeStruct((n_sc, FH), jnp.float32),
            jax.ShapeDtypeStruct((n_sc, FH), jnp.float32),
        ),
        mesh=_mesh(),
        compiler_params=_SC_PARAMS,
        scratch_types=[
            pltpu.VMEM_SHARED((n_sc, FH), jnp.float32),
            pltpu.VMEM((NBATCH, BATCH), jnp.int32),
            pltpu.VMEM((NBATCH, BATCH), jnp.int32),
            pltpu.VMEM((NBATCH, BATCH), jnp.int32),
            pltpu.VMEM((NBATCH, BATCH), jnp.int32),
            pltpu.VMEM((NBATCH, BATCH), jnp.int32),
            pltpu.VMEM((NBATCH, BATCH), jnp.int32),
            pltpu.VMEM((NBATCH, BATCH), jnp.int32),
            pltpu.VMEM((NBATCH, BATCH), jnp.int32),
            pltpu.VMEM((SB, FH), jnp.float32),
            pltpu.VMEM((SB, FH), jnp.float32),
            pltpu.SemaphoreType.DMA,
            pltpu.SemaphoreType.DMA,
            pltpu.SemaphoreType.DMA,
            pltpu.SemaphoreType.DMA,
        ],
    )
    def edge_pass(
        table, src_hbm, dst_hbm, out_lo, out_hi,
        acc_sh, srcA, dstA, gidxA, sidxA, srcB, dstB, gidxB, sidxB,
        rowsA, rowsB, sem_i, sem_g, sem_s, sem_o,
    ):
        cid = lax.axis_index("c")
        sid = lax.axis_index("s")
        qoff = off + cid

        zeros16 = jnp.zeros((LANES,), jnp.float32)

        def fill0(i, _):
            rowsA[i, pl.ds(0, LANES)] = zeros16
            return 0

        lax.fori_loop(0, SB, fill0, 0)
        for k in range(d_r // zc):
            pltpu.sync_copy(
                rowsA.at[pl.ds(0, zc)],
                acc_sh.at[pl.ds(sid * d_r + k * zc, zc)],
            )
        plsc.subcore_barrier()

        row_base = sid * (sb_per_tile * NBATCH)
        nsb = sb_per_tile

        def compute_gidx(src_v, dst_v, gidx_v, sidx_v):
            for j in range(NBATCH):
                for q in range(BATCH // LANES):
                    s16 = src_v[j, pl.ds(q * LANES, LANES)]
                    gidx_v[j, pl.ds(q * LANES, LANES)] = s16 * k_int + qoff
                    sidx_v[j, pl.ds(q * LANES, LANES)] = dst_v[j, pl.ds(q * LANES, LANES)]

        def fire_gathers(gidx_v, rows_v):
            for j in range(NBATCH):
                pltpu.async_copy(
                    table.at[gidx_v.at[j]],
                    rows_v.at[pl.ds(j * BATCH, BATCH)],
                    sem_g,
                )

        def wait_gathers(gidx_v, rows_v):
            for j in range(NBATCH):
                pltpu.make_async_copy(
                    table.at[gidx_v.at[j]],
                    rows_v.at[pl.ds(j * BATCH, BATCH)],
                    sem_g,
                ).wait()

        def fire_scatters(dst_v, rows_v):
            for j in range(NBATCH):
                pltpu.async_copy(
                    rows_v.at[pl.ds(j * BATCH, BATCH)],
                    acc_sh.at[dst_v.at[j]],
                    sem_s,
                    add=True,
                )

        def wait_scatters(dst_v, rows_v):
            for j in range(NBATCH):
                pltpu.make_async_copy(
                    rows_v.at[pl.ds(j * BATCH, BATCH)],
                    acc_sh.at[dst_v.at[j]],
                    sem_s,
                ).wait()

        pltpu.sync_copy(src_hbm.at[pl.ds(row_base, NBATCH)], srcA)
        pltpu.sync_copy(dst_hbm.at[pl.ds(row_base, NBATCH)], dstA)
        compute_gidx(srcA, dstA, gidxA, sidxA)

        def one_iter(g, cur_gidx, cur_sidx, cur_src, cur_dst, cur_rows,
                     prv_gidx, prv_sidx, prv_src, prv_dst, prv_rows):
            @pl.when(g > 0)
            def _():
                pltpu.make_async_copy(
                    src_hbm.at[pl.ds(row_base, NBATCH)], cur_src, sem_i
                ).wait()
                pltpu.make_async_copy(
                    dst_hbm.at[pl.ds(row_base, NBATCH)], cur_dst, sem_i
                ).wait()

            @pl.when(g > 1)
            def _():
                wait_scatters(cur_sidx, cur_rows)

            @pl.when(g > 0)
            def _():
                compute_gidx(cur_src, cur_dst, cur_gidx, cur_sidx)

            fire_gathers(cur_gidx, cur_rows)

            @pl.when(g > 0)
            def _():
                wait_gathers(prv_gidx, prv_rows)
                fire_scatters(prv_sidx, prv_rows)

            @pl.when(g + 1 < nsb)
            def _():
                rb1 = row_base + (g + 1) * NBATCH
                pltpu.async_copy(src_hbm.at[pl.ds(rb1, NBATCH)], prv_src, sem_i)
                pltpu.async_copy(dst_hbm.at[pl.ds(rb1, NBATCH)], prv_dst, sem_i)

        def body(g, _):
            @pl.when(g % 2 == 0)
            def _():
                one_iter(g, gidxA, sidxA, srcA, dstA, rowsA,
                         gidxB, sidxB, srcB, dstB, rowsB)

            @pl.when(g % 2 == 1)
            def _():
                one_iter(g, gidxB, sidxB, srcB, dstB, rowsB,
                         gidxA, sidxA, srcA, dstA, rowsA)

            return 0

        lax.fori_loop(0, nsb, body, 0)
        if (nsb - 1) % 2 == 0:
            lgidx, lsidx, lrows = gidxA, sidxA, rowsA
            osidx, orows = sidxB, rowsB
        else:
            lgidx, lsidx, lrows = gidxB, sidxB, rowsB
            osidx, orows = sidxA, rowsA
        wait_gathers(lgidx, lrows)
        fire_scatters(lsidx, lrows)
        if nsb > 1:
            wait_scatters(osidx, orows)
        wait_scatters(lsidx, lrows)
        plsc.subcore_barrier()

        def copy_out(out_hbm):
            nchunks = out_rows // oc
            for k in range(nchunks):
                rbuf = rowsA if k % 2 == 0 else rowsB
                if k >= 2:
                    pltpu.make_async_copy(
                        rbuf.at[pl.ds(0, oc)],
                        out_hbm.at[pl.ds(sid * out_rows, oc)],
                        sem_o,
                    ).wait()
                pltpu.sync_copy(
                    acc_sh.at[pl.ds(sid * out_rows + k * oc, oc)],
                    rbuf.at[pl.ds(0, oc)],
                )
                pltpu.async_copy(
                    rbuf.at[pl.ds(0, oc)],
                    out_hbm.at[pl.ds(sid * out_rows + k * oc, oc)],
                    sem_o,
                )
            for k in range(min(2, nchunks)):
                rbuf = rowsA if (nchunks - 2 + k) % 2 == 0 else rowsB
                pltpu.make_async_copy(
                    rbuf.at[pl.ds(0, oc)],
                    out_hbm.at[pl.ds(sid * out_rows, oc)],
                    sem_o,
                ).wait()

        @pl.when(cid == 0)
        def _():
            copy_out(out_lo)

        @pl.when(cid == 1)
        def _():
            copy_out(out_hi)

    return edge_pass


def _full(rows, cols):
    return pl.BlockSpec((rows, cols), lambda i: (0, 0))


def _blk(rows, cols):
    return pl.BlockSpec((rows, cols), lambda i: (i, 0))


def _kdinv_body(d0_ref, d1_ref, b16_ref, r16_ref):
    dinv = lax.rsqrt(d0_ref[...] + d1_ref[...] + 1.0)
    r16_ref[...] = jnp.dot(dinv, b16_ref[...],
                           preferred_element_type=jnp.float32, precision=lax.Precision.HIGHEST)


def _widen(r16, rep):
    pieces = []
    for a in range(8):
        t = r16[:, 16 * a : 16 * (a + 1)]
        pieces.extend([t] * rep)
    return jnp.concatenate(pieces, axis=1)


def _ka_body(x_ref, w_ref, r16_ref, g0_ref):
    g0_ref[...] = _widen(r16_ref[...], 2) * jnp.dot(
        x_ref[...], w_ref[...], preferred_element_type=jnp.float32, precision=lax.Precision.HIGHEST)


def _kb_body(alo_ref, ahi_ref, r16_ref, g0_ref, b_ref, w_ref,
             blo_ref, bhi_ref, h1_ref, g1_ref):
    r16 = r16_ref[...]
    accw = (jnp.dot(r16 * alo_ref[...], blo_ref[...],
                    preferred_element_type=jnp.float32, precision=lax.Precision.HIGHEST)
            + jnp.dot(r16 * ahi_ref[...], bhi_ref[...],
                      preferred_element_type=jnp.float32, precision=lax.Precision.HIGHEST))
    r32 = _widen(r16, 2)
    h1 = jnp.maximum(accw + r32 * g0_ref[...] + b_ref[...], 0.0)
    h1_ref[...] = h1
    g1_ref[...] = jnp.dot(r32 * h1, w_ref[...],
                          preferred_element_type=jnp.float32, precision=lax.Precision.HIGHEST)


def _kc_body(alo_ref, ahi_ref, r16_ref, g1_ref, h1_ref, b_ref,
             w_ref, blo_ref, bhi_ref, g2_ref):
    r16 = r16_ref[...]
    accw = (jnp.dot(r16 * alo_ref[...], blo_ref[...],
                    preferred_element_type=jnp.float32, precision=lax.Precision.HIGHEST)
            + jnp.dot(r16 * ahi_ref[...], bhi_ref[...],
                      preferred_element_type=jnp.float32, precision=lax.Precision.HIGHEST))
    r32 = _widen(r16, 2)
    h2 = (jnp.maximum(accw + r32 * g1_ref[...] + b_ref[...], 0.0)
          + h1_ref[...])
    g2_ref[...] = jnp.dot(r32 * h2, w_ref[...],
                          preferred_element_type=jnp.float32, precision=lax.Precision.HIGHEST)


def _kd_body(a0_ref, a1_ref, a2_ref, a3_ref, r16_ref, g2_ref,
             b_ref, p0_ref, p1_ref, p2_ref, p3_ref, out_ref):
    r16 = r16_ref[...]
    acc = jnp.dot(r16 * a0_ref[...], p0_ref[...],
                  preferred_element_type=jnp.float32, precision=lax.Precision.HIGHEST)
    acc = acc + jnp.dot(r16 * a1_ref[...], p1_ref[...],
                        preferred_element_type=jnp.float32, precision=lax.Precision.HIGHEST)
    acc = acc + jnp.dot(r16 * a2_ref[...], p2_ref[...],
                        preferred_element_type=jnp.float32, precision=lax.Precision.HIGHEST)
    acc = acc + jnp.dot(r16 * a3_ref[...], p3_ref[...],
                        preferred_element_type=jnp.float32, precision=lax.Precision.HIGHEST)
    out_ref[...] = acc + _widen(r16, 4) * g2_ref[...] + b_ref[...]


def kernel(x, edge_index, W0, b0, W1, b1, W_out, b_out):
    n, dfeat = x.shape
    e = edge_index.shape[1]
    nh = W0.shape[1]
    nclass = W_out.shape[1]
    grid_n = -(-n // _TC_R)
    n_sc = grid_n * _TC_R
    grid = (grid_n,)

    src = edge_index[0]
    dst = edge_index[1]
    e_pad = _round_up(e, NC * NS * SB)
    pad = e_pad - e
    src_p = jnp.concatenate([src, jnp.zeros((pad,), jnp.int32)]).reshape(-1, BATCH)
    dst_p = jnp.concatenate([dst, jnp.full((pad,), jnp.int32(n))]).reshape(-1, BATCH)

    m = np.arange(128)
    B16 = (m[:, None] == (np.arange(16 * 128) // 16)[None, :]).astype(np.float32)
    Blo = ((32 * (m // 16) + m % 16)[:, None]
           == np.arange(256)[None, :]).astype(np.float32)
    Bhi = ((32 * (m // 16) + 16 + m % 16)[:, None]
           == np.arange(256)[None, :]).astype(np.float32)
    B64 = [((64 * (m // 16) + 16 * j + m % 16)[:, None]
            == np.arange(512)[None, :]).astype(np.float32) for j in range(4)]

    W0bd = jnp.kron(jnp.eye(8, dtype=jnp.float32), W0)
    W1bd = jnp.kron(jnp.eye(8, dtype=jnp.float32), W1)
    Wobd = jnp.kron(jnp.eye(8, dtype=jnp.float32), W_out)
    b0w = jnp.tile(b0, 8)[None, :]
    b1w = jnp.tile(b1, 8)[None, :]
    bow = jnp.tile(b_out, 8)[None, :]

    deg_pp = _make_deg_kernel(n_sc, e_pad)(dst_p)
    d0 = deg_pp[:n_sc].reshape(n_sc // 128, 128)
    d1 = deg_pp[n_sc:].reshape(n_sc // 128, 128)

    pk1 = n_sc // 128
    pkf = n_sc * FH // 128

    kdinv = pl.pallas_call(
        _kdinv_body,
        grid=grid,
        in_specs=[_blk(16, 128), _blk(16, 128), _full(128, 2048)],
        out_specs=_blk(16, 2048),
        out_shape=jax.ShapeDtypeStruct((pk1, 2048), jnp.float32),
    )
    r16w = kdinv(d0, d1, B16)
    rep16 = r16w.reshape(pkf, 128)

    ka = pl.pallas_call(
        _ka_body,
        grid=grid,
        in_specs=[_blk(256, 1024), _full(1024, 256), _blk(256, 128)],
        out_specs=_blk(256, 256),
        out_shape=jax.ShapeDtypeStruct((n_sc // 8, 256), jnp.float32),
    )
    g0w = ka(x.reshape(n // 8, 8 * dfeat), W0bd, rep16)

    ep2 = _make_edge_pass(n_sc, e_pad, 2, 0)
    a0lo, a0hi = ep2(g0w.reshape(2 * n_sc, FH), src_p, dst_p)

    kb = pl.pallas_call(
        _kb_body,
        grid=grid,
        in_specs=[_blk(256, 128), _blk(256, 128), _blk(256, 128),
                  _blk(256, 256), _full(1, 256),
                  _full(256, 256), _full(128, 256), _full(128, 256)],
        out_specs=[_blk(256, 256), _blk(256, 256)],
        out_shape=[
            jax.ShapeDtypeStruct((n_sc // 8, 256), jnp.float32),
            jax.ShapeDtypeStruct((n_sc // 8, 256), jnp.float32),
        ],
    )
    h1w, g1w = kb(a0lo.reshape(pkf, 128), a0hi.reshape(pkf, 128), rep16,
                  g0w, b0w, W1bd, Blo, Bhi)

    a1lo, a1hi = ep2(g1w.reshape(2 * n_sc, FH), src_p, dst_p)

    kc = pl.pallas_call(
        _kc_body,
        grid=grid,
        in_specs=[_blk(256, 128), _blk(256, 128), _blk(256, 128),
                  _blk(256, 256), _blk(256, 256),
                  _full(1, 256), _full(256, 512), _full(128, 256),
                  _full(128, 256)],
        out_specs=_blk(256, 512),
        out_shape=jax.ShapeDtypeStruct((n_sc // 8, 512), jnp.float32),
    )
    g2w = kc(a1lo.reshape(pkf, 128), a1hi.reshape(pkf, 128), rep16,
             g1w, h1w, b1w, Wobd, Blo, Bhi)

    g2_tbl = g2w.reshape(4 * n_sc, FH)
    ep4a = _make_edge_pass(n_sc, e_pad, 4, 0)
    ep4b = _make_edge_pass(n_sc, e_pad, 4, 2)
    a2q0, a2q1 = ep4a(g2_tbl, src_p, dst_p)
    a2q2, a2q3 = ep4b(g2_tbl, src_p, dst_p)

    kd = pl.pallas_call(
        _kd_body,
        grid=grid,
        in_specs=[_blk(256, 128)] * 4 + [_blk(256, 128), _blk(256, 512),
                  _full(1, 512)]
                 + [_full(128, 512)] * 4,
        out_specs=_blk(256, 512),
        out_shape=jax.ShapeDtypeStruct((n_sc // 8, 512), jnp.float32),
    )
    outw = kd(a2q0.reshape(pkf, 128), a2q1.reshape(pkf, 128),
              a2q2.reshape(pkf, 128), a2q3.reshape(pkf, 128),
              rep16, g2w, bow, B64[0], B64[1], B64[2], B64[3])
    return outw.reshape(n_sc, nclass)[:n]
